# Initial kernel scaffold; baseline (speedup 1.0000x reference)
import jax
import jax.numpy as jnp
from jax.experimental import pallas as pl


def kernel(token_ids, segment_ids, W_word, W_pos, W_seg, W_type, gamma, beta):
    raise NotImplementedError("write your pallas kernel here")


# speedup vs baseline: 2.3331x; 2.3331x over previous
import functools

import jax
import jax.numpy as jnp
from jax import lax
from jax.experimental import pallas as pl
from jax.experimental.pallas import tpu as pltpu
from jax.experimental.pallas import tpu_sc as plsc

B = 1024
S = 200
HID = 128
VOCAB = 100000
TYPE_V = 6
L = 16
NW = 32
TOK_PER_W = B * S // NW
ROWS_PER_CHUNK = 2
CHUNK = ROWS_PER_CHUNK * S
CHUNKS_PER_W = TOK_PER_W // CHUNK
IDXW = 50
NJ = CHUNK // IDXW

_mesh = plsc.VectorSubcoreMesh(core_axis_name="c", subcore_axis_name="s")


@functools.partial(
    pl.kernel,
    out_type=jax.ShapeDtypeStruct((B * S, HID), jnp.float32),
    mesh=_mesh,
    compiler_params=pltpu.CompilerParams(needs_layout_passes=False),
    scratch_types=[
        pltpu.VMEM((NJ, IDXW), jnp.int32),
        pltpu.VMEM((CHUNK,), jnp.int32),
        pltpu.VMEM((CHUNK, HID), jnp.float32),
        pltpu.VMEM((S, HID), jnp.float32),
        pltpu.VMEM((TYPE_V, HID), jnp.float32),
        pltpu.VMEM((HID,), jnp.float32),
        pltpu.VMEM((HID,), jnp.float32),
        pltpu.SemaphoreType.DMA,
    ],
)
def _emb_ln(tok2d_hbm, tok1d_hbm, W_word_hbm, base_hbm, type_hbm, gamma_hbm,
            beta_hbm, out_hbm, idx_v, tok_s, rows_v, base_v, type_v, gamma_v,
            beta_v, sem):
    wid = lax.axis_index("s") * 2 + lax.axis_index("c")
    pltpu.sync_copy(base_hbm, base_v)
    pltpu.sync_copy(type_hbm, type_v)
    pltpu.sync_copy(gamma_hbm, gamma_v)
    pltpu.sync_copy(beta_hbm, beta_v)

    iotas = [lax.iota(jnp.int32, L) + L * j for j in range(HID // L)]

    @pl.loop(0, CHUNKS_PER_W)
    def _chunk(c):
        tok0 = wid * TOK_PER_W + c * CHUNK
        r0 = pl.multiple_of(tok0 // IDXW, 8)
        pltpu.sync_copy(tok2d_hbm.at[pl.ds(r0, NJ)], idx_v)
        pltpu.sync_copy(tok1d_hbm.at[pl.ds(tok0, CHUNK)], tok_s)
        cps = [
            pltpu.async_copy(W_word_hbm.at[idx_v.at[j]],
                             rows_v.at[pl.ds(IDXW * j, IDXW)], sem)
            for j in range(NJ)
        ]
        for cp in cps:
            cp.wait()

        @pl.loop(0, CHUNK // L)
        def _grp(g):
            tvec = lax.rem(tok_s[pl.ds(L * g, L)], TYPE_V)
            for t in range(L):
                i = L * g + t
                s = lax.rem(i, S)
                tmod = tvec[t]
                accs = []
                for j in range(HID // L):
                    w = rows_v[i, pl.ds(L * j, L)]
                    b = base_v[s, pl.ds(L * j, L)]
                    ty = type_v[tmod, pl.ds(L * j, L)]
                    accs.append(w + b + ty)
                tot = accs[0]
                for j in range(1, HID // L):
                    tot = tot + accs[j]
                total = jnp.sum(tot)
                sq = accs[0] * accs[0]
                for j in range(1, HID // L):
                    sq = sq + accs[j] * accs[j]
                totalsq = jnp.sum(sq)
                mean = total * (1.0 / HID)
                var = totalsq * (1.0 / HID) - mean * mean
                rv = jnp.full((L,), var + 1e-12, dtype=jnp.float32)
                bi = plsc.bitcast(rv, jnp.int32)
                bi = 0x5F3759DF - lax.shift_right_logical(bi, 1)
                y = plsc.bitcast(bi, jnp.float32)
                for _ in range(3):
                    y = y * (1.5 - 0.5 * rv * y * y)
                for j in range(HID // L):
                    gm = gamma_v[pl.ds(L * j, L)]
                    be = beta_v[pl.ds(L * j, L)]
                    rows_v[i, pl.ds(L * j, L)] = (accs[j] - mean) * y * gm + be

        pltpu.sync_copy(rows_v, out_hbm.at[pl.ds(tok0, CHUNK)])


def kernel(token_ids, segment_ids, W_word, W_pos, W_seg, W_type, gamma, beta):
    del segment_ids
    base = W_pos[:S] + W_seg[0][None, :]
    tok1d = token_ids.reshape(B * S)
    tok2d = tok1d.reshape(B * S // IDXW, IDXW)
    out = _emb_ln(tok2d, tok1d, W_word, base, W_type, gamma, beta)
    return out.reshape(B, S, HID)


# speedup vs baseline: 5.9945x; 2.5693x over previous
import functools

import jax
import jax.numpy as jnp
from jax import lax
from jax.experimental import pallas as pl
from jax.experimental.pallas import tpu as pltpu
from jax.experimental.pallas import tpu_sc as plsc

B = 1024
S = 200
HID = 128
VOCAB = 100000
TYPE_V = 6
L = 16
NW = 32
TOK_PER_W = B * S // NW
ROWS_PER_CHUNK = 2
CHUNK = ROWS_PER_CHUNK * S
CHUNKS_PER_W = TOK_PER_W // CHUNK
IDXW = 50
NJ = CHUNK // IDXW
CIDXW = 80
NCJ = CHUNK // CIDXW
NG = CHUNK // L

_mesh = plsc.VectorSubcoreMesh(core_axis_name="c", subcore_axis_name="s")


@functools.partial(
    pl.kernel,
    out_type=jax.ShapeDtypeStruct((B * S, HID), jnp.float32),
    mesh=_mesh,
    compiler_params=pltpu.CompilerParams(needs_layout_passes=False),
    scratch_types=[
        pltpu.VMEM((NJ, IDXW), jnp.int32),
        pltpu.VMEM((CHUNK,), jnp.int32),
        pltpu.VMEM((NCJ, CIDXW), jnp.int32),
        pltpu.VMEM((CHUNK, HID), jnp.float32),
        pltpu.VMEM((CHUNK, HID), jnp.float32),
        pltpu.VMEM((NG, L), jnp.int32),
        pltpu.VMEM((HID,), jnp.float32),
        pltpu.VMEM((HID,), jnp.float32),
        pltpu.SemaphoreType.DMA,
    ],
)
def _emb_ln(tok2d_hbm, tok1d_hbm, W_word_hbm, comb_hbm, pos6_hbm, gamma_hbm,
            beta_hbm, out_hbm, idx_v, tok_v, idx2_v, rows_v, add_v, pos6_v,
            gamma_v, beta_v, sem):
    wid = lax.axis_index("s") * 2 + lax.axis_index("c")
    pltpu.sync_copy(pos6_hbm, pos6_v)
    pltpu.sync_copy(gamma_hbm, gamma_v)
    pltpu.sync_copy(beta_hbm, beta_v)

    @pl.loop(0, CHUNKS_PER_W)
    def _chunk(c):
        tok0 = wid * TOK_PER_W + c * CHUNK
        r0 = pl.multiple_of(tok0 // IDXW, 8)
        pltpu.sync_copy(tok2d_hbm.at[pl.ds(r0, NJ)], idx_v)
        pltpu.sync_copy(tok1d_hbm.at[pl.ds(tok0, CHUNK)], tok_v)
        word_cps = [
            pltpu.async_copy(W_word_hbm.at[idx_v.at[j]],
                             rows_v.at[pl.ds(IDXW * j, IDXW)], sem)
            for j in range(NJ)
        ]
        for r in range(NCJ):
            for k in range(CIDXW // L):
                g = (CIDXW // L) * r + k
                tv = tok_v[pl.ds(CIDXW * r + L * k, L)]
                idx2_v[r, pl.ds(L * k, L)] = pos6_v[g] + lax.rem(tv, TYPE_V)
        comb_cps = [
            pltpu.async_copy(comb_hbm.at[idx2_v.at[r]],
                             add_v.at[pl.ds(CIDXW * r, CIDXW)], sem)
            for r in range(NCJ)
        ]
        for cp in word_cps + comb_cps:
            cp.wait()

        @pl.loop(0, NG)
        def _grp(g):
            for t in range(L):
                i = L * g + t
                accs = []
                for j in range(HID // L):
                    w = rows_v[i, pl.ds(L * j, L)]
                    a = add_v[i, pl.ds(L * j, L)]
                    accs.append(w + a)
                tot = accs[0]
                for j in range(1, HID // L):
                    tot = tot + accs[j]
                total = jnp.sum(tot)
                sq = accs[0] * accs[0]
                for j in range(1, HID // L):
                    sq = sq + accs[j] * accs[j]
                totalsq = jnp.sum(sq)
                mean = total * (1.0 / HID)
                var = totalsq * (1.0 / HID) - mean * mean
                rv = jnp.full((L,), var + 1e-12, dtype=jnp.float32)
                bi = plsc.bitcast(rv, jnp.int32)
                bi = 0x5F3759DF - lax.shift_right_logical(bi, 1)
                y = plsc.bitcast(bi, jnp.float32)
                for _ in range(3):
                    y = y * (1.5 - 0.5 * rv * y * y)
                for j in range(HID // L):
                    gm = gamma_v[pl.ds(L * j, L)]
                    be = beta_v[pl.ds(L * j, L)]
                    rows_v[i, pl.ds(L * j, L)] = (accs[j] - mean) * y * gm + be

        pltpu.sync_copy(rows_v, out_hbm.at[pl.ds(tok0, CHUNK)])


def kernel(token_ids, segment_ids, W_word, W_pos, W_seg, W_type, gamma, beta):
    del segment_ids
    comb = (W_pos[:S, None, :] + W_seg[0][None, None, :]
            + W_type[None, :, :]).reshape(S * TYPE_V, HID)
    pos6 = (TYPE_V * (jnp.arange(CHUNK, dtype=jnp.int32) % S)).reshape(NG, L)
    tok1d = token_ids.reshape(B * S)
    tok2d = tok1d.reshape(B * S // IDXW, IDXW)
    out = _emb_ln(tok2d, tok1d, W_word, comb, pos6, gamma, beta)
    return out.reshape(B, S, HID)


# speedup vs baseline: 6.8074x; 1.1356x over previous
import functools

import jax
import jax.numpy as jnp
from jax import lax
from jax.experimental import pallas as pl
from jax.experimental.pallas import tpu as pltpu
from jax.experimental.pallas import tpu_sc as plsc

B = 1024
S = 200
HID = 128
VOCAB = 100000
TYPE_V = 6
L = 16
NW = 32
TOK_PER_W = B * S // NW
CHUNK = 160
NCHUNK = TOK_PER_W // CHUNK
IDXW = 20
NJ = CHUNK // IDXW
CIDXW = 80
NCJ = CHUNK // CIDXW
KPR = CIDXW // L
NGC = CHUNK // L
PERIOD = 400 // L

_mesh = plsc.VectorSubcoreMesh(core_axis_name="c", subcore_axis_name="s")

_f32 = jnp.float32
_i32 = jnp.int32


@functools.partial(
    pl.kernel,
    out_type=jax.ShapeDtypeStruct((B * S, HID), _f32),
    mesh=_mesh,
    compiler_params=pltpu.CompilerParams(needs_layout_passes=False),
    scratch_types=[
        [pltpu.VMEM((NJ, IDXW), _i32)] * 2,
        [pltpu.VMEM((CHUNK,), _i32)] * 2,
        [pltpu.VMEM((NCJ, CIDXW), _i32)] * 2,
        [pltpu.VMEM((CHUNK, HID), _f32)] * 2,
        [pltpu.VMEM((CHUNK, HID), _f32)] * 2,
        pltpu.VMEM((PERIOD, L), _i32),
        pltpu.VMEM((HID,), _f32),
        pltpu.VMEM((HID,), _f32),
        [pltpu.SemaphoreType.DMA] * 2,
        [pltpu.SemaphoreType.DMA] * 2,
        [pltpu.SemaphoreType.DMA] * 2,
    ],
)
def _emb_ln(tok2d_hbm, tok1d_hbm, W_word_hbm, comb_hbm, pos6_hbm, gamma_hbm,
            beta_hbm, out_hbm, idx_v, tok_v, idx2_v, rows_v, add_v, pos6_v,
            gamma_v, beta_v, isem, gsem, osem):
    wid = lax.axis_index("s") * 2 + lax.axis_index("c")
    pltpu.sync_copy(pos6_hbm, pos6_v)
    pltpu.sync_copy(gamma_hbm, gamma_v)
    pltpu.sync_copy(beta_hbm, beta_v)

    def drain(src, dst, sem):
        pltpu.make_async_copy(src, dst, sem).wait()

    def fire_idx(c, p):
        tok0 = wid * TOK_PER_W + c * CHUNK
        r0 = pl.multiple_of(tok0 // IDXW, 8)
        pltpu.async_copy(tok2d_hbm.at[pl.ds(r0, NJ)], idx_v[p], isem[p])
        pltpu.async_copy(tok1d_hbm.at[pl.ds(tok0, CHUNK)], tok_v[p], isem[p])

    def fire_gathers(c, p):
        drain(tok2d_hbm.at[pl.ds(0, NJ)], idx_v[p], isem[p])
        drain(tok1d_hbm.at[pl.ds(0, CHUNK)], tok_v[p], isem[p])
        for r in range(NCJ):
            for k in range(KPR):
                g = lax.rem(c * NGC + KPR * r + k, PERIOD)
                tv = tok_v[p][pl.ds(CIDXW * r + L * k, L)]
                idx2_v[p][r, pl.ds(L * k, L)] = pos6_v[g] + lax.rem(tv, TYPE_V)
        for j in range(NJ):
            pltpu.async_copy(W_word_hbm.at[idx_v[p].at[j]],
                             rows_v[p].at[pl.ds(IDXW * j, IDXW)], gsem[p])
        for r in range(NCJ):
            pltpu.async_copy(comb_hbm.at[idx2_v[p].at[r]],
                             add_v[p].at[pl.ds(CIDXW * r, CIDXW)], gsem[p])

    def compute(c, p):
        drain(out_hbm.at[pl.ds(0, CHUNK)], rows_v[p], gsem[p])
        drain(out_hbm.at[pl.ds(0, CHUNK)], add_v[p], gsem[p])

        @pl.loop(0, NGC)
        def _grp(g):
            for t in range(L):
                i = L * g + t
                accs = []
                for j in range(HID // L):
                    w = rows_v[p][i, pl.ds(L * j, L)]
                    a = add_v[p][i, pl.ds(L * j, L)]
                    accs.append(w + a)
                tot = accs[0]
                for j in range(1, HID // L):
                    tot = tot + accs[j]
                total = jnp.sum(tot)
                sq = accs[0] * accs[0]
                for j in range(1, HID // L):
                    sq = sq + accs[j] * accs[j]
                totalsq = jnp.sum(sq)
                mean = total * (1.0 / HID)
                var = totalsq * (1.0 / HID) - mean * mean
                rv = jnp.full((L,), var + 1e-12, dtype=_f32)
                bi = plsc.bitcast(rv, _i32)
                bi = 0x5F3759DF - lax.shift_right_logical(bi, 1)
                y = plsc.bitcast(bi, _f32)
                for _ in range(2):
                    y = y * (1.5 - 0.5 * rv * y * y)
                for j in range(HID // L):
                    gm = gamma_v[pl.ds(L * j, L)]
                    be = beta_v[pl.ds(L * j, L)]
                    rows_v[p][i, pl.ds(L * j, L)] = (accs[j] - mean) * y * gm + be

        tok0 = wid * TOK_PER_W + c * CHUNK
        pltpu.async_copy(rows_v[p], out_hbm.at[pl.ds(tok0, CHUNK)], osem[p])

    def drain_out(p):
        drain(out_hbm.at[pl.ds(0, CHUNK)], rows_v[p], osem[p])

    fire_idx(0, 0)
    fire_gathers(0, 0)
    fire_idx(1, 1)
    fire_gathers(1, 1)

    @pl.loop(0, NCHUNK - 2, step=2)
    def _body(c):
        compute(c, 0)
        fire_idx(c + 2, 0)
        compute(c + 1, 1)
        fire_idx(c + 3, 1)
        drain_out(0)
        fire_gathers(c + 2, 0)
        drain_out(1)
        fire_gathers(c + 3, 1)

    compute(NCHUNK - 2, 0)
    compute(NCHUNK - 1, 1)
    drain_out(0)
    drain_out(1)


def kernel(token_ids, segment_ids, W_word, W_pos, W_seg, W_type, gamma, beta):
    del segment_ids
    comb = (W_pos[:S, None, :] + W_seg[0][None, None, :]
            + W_type[None, :, :]).reshape(S * TYPE_V, HID)
    pos6 = (TYPE_V * (jnp.arange(PERIOD * L, dtype=_i32) % S)).reshape(PERIOD, L)
    tok1d = token_ids.reshape(B * S)
    tok2d = tok1d.reshape(B * S // IDXW, IDXW)
    out = _emb_ln(tok2d, tok1d, W_word, comb, pos6, gamma, beta)
    return out.reshape(B, S, HID)
